# Initial kernel scaffold; baseline (speedup 1.0000x reference)
import jax
import jax.numpy as jnp
from jax.experimental import pallas as pl


def kernel(x, edge_index, mask, W, b):
    raise NotImplementedError("write your pallas kernel here")


# speedup vs baseline: 15.5011x; 15.5011x over previous
import functools

import jax
import jax.numpy as jnp
from jax import lax
from jax.experimental import pallas as pl
from jax.experimental.pallas import tpu as pltpu
from jax.experimental.pallas import tpu_sc as plsc

N = 10000
E = 320000
D = 128

NC = 2
NS = 16
NW = NC * NS
EPW = E // NW
CH = 80
NCHUNK = EPW // CH
NPAD = 10240
RCH = NPAD // (NS * CH)

_mesh = plsc.VectorSubcoreMesh(
    core_axis_name="c", subcore_axis_name="s", num_cores=NC, num_subcores=NS
)


@functools.partial(
    pl.kernel,
    out_type=jax.ShapeDtypeStruct((NC, NPAD, 16), jnp.float32),
    mesh=_mesh,
    scratch_types=[
        pltpu.VMEM((CH,), jnp.int32),
        pltpu.VMEM((CH, 16), jnp.float32),
        pltpu.VMEM((CH, 16), jnp.float32),
        pltpu.VMEM_SHARED((NPAD, 16), jnp.float32),
    ],
)
def _deg_kernel(dst_hbm, out_hbm, didx_v, ones_v, zeros_v, hist_sh):
    c = lax.axis_index("c")
    s = lax.axis_index("s")
    wid = s * NC + c

    @pl.loop(0, CH)
    def _fill(r):
        ones_v[r] = jnp.ones((16,), jnp.float32)
        zeros_v[r] = jnp.zeros((16,), jnp.float32)

    @pl.loop(0, RCH)
    def _zero(k):
        pltpu.sync_copy(zeros_v, hist_sh.at[pl.ds((s * RCH + k) * CH, CH)])

    plsc.subcore_barrier()

    base = wid * EPW

    @pl.loop(0, NCHUNK)
    def _accum(i):
        pltpu.sync_copy(dst_hbm.at[pl.ds(base + i * CH, CH)], didx_v)
        pltpu.sync_copy(ones_v, hist_sh.at[didx_v], add=True)

    plsc.subcore_barrier()

    @pl.loop(0, RCH)
    def _writeback(k):
        r0 = (s * RCH + k) * CH
        pltpu.sync_copy(hist_sh.at[pl.ds(r0, CH)], ones_v)
        pltpu.sync_copy(ones_v, out_hbm.at[c, pl.ds(r0, CH)])


_RB = 2000


def _lin_body(x_ref, m_ref, h0_ref, h1_ref, w_ref, g_ref, s_ref):
    deg = 1.0 + h0_ref[...] + h1_ref[...]
    sv = m_ref[...] * lax.rsqrt(deg)
    s_ref[...] = sv
    g_ref[...] = jnp.dot(
        x_ref[...] * sv, w_ref[...], preferred_element_type=jnp.float32
    )


_linear = pl.pallas_call(
    _lin_body,
    grid=(N // _RB,),
    in_specs=[
        pl.BlockSpec((_RB, D), lambda i: (i, 0)),
        pl.BlockSpec((_RB, 1), lambda i: (i, 0)),
        pl.BlockSpec((_RB, 1), lambda i: (i, 0)),
        pl.BlockSpec((_RB, 1), lambda i: (i, 0)),
        pl.BlockSpec((D, D), lambda i: (0, 0)),
    ],
    out_specs=[
        pl.BlockSpec((_RB, D), lambda i: (i, 0)),
        pl.BlockSpec((_RB, 1), lambda i: (i, 0)),
    ],
    out_shape=[
        jax.ShapeDtypeStruct((N, D), jnp.float32),
        jax.ShapeDtypeStruct((N, 1), jnp.float32),
    ],
)


@functools.partial(
    pl.kernel,
    out_type=jax.ShapeDtypeStruct((NC, NPAD, D), jnp.float32),
    mesh=_mesh,
    scratch_types=[
        pltpu.VMEM((CH,), jnp.int32),
        pltpu.VMEM((CH,), jnp.int32),
        pltpu.VMEM((CH, D), jnp.float32),
        pltpu.VMEM_SHARED((NPAD, D), jnp.float32),
        pltpu.SemaphoreType.DMA,
    ],
)
def _edge_kernel(src_hbm, dst_hbm, g_hbm, out_hbm, sidx_v, didx_v, rows_v, acc_sh, sem):
    c = lax.axis_index("c")
    s = lax.axis_index("s")
    wid = s * NC + c

    @pl.loop(0, CH)
    def _zero_rows(r):
        for j in range(D // 16):
            rows_v[r, pl.ds(j * 16, 16)] = jnp.zeros((16,), jnp.float32)

    @pl.loop(0, RCH)
    def _zero_acc(k):
        pltpu.sync_copy(rows_v, acc_sh.at[pl.ds((s * RCH + k) * CH, CH)])

    plsc.subcore_barrier()

    base = wid * EPW

    @pl.loop(0, NCHUNK)
    def _accum(i):
        off = base + i * CH
        pltpu.sync_copy(src_hbm.at[pl.ds(off, CH)], sidx_v)
        pltpu.sync_copy(dst_hbm.at[pl.ds(off, CH)], didx_v)
        pltpu.async_copy(g_hbm.at[sidx_v], rows_v, sem).wait()
        pltpu.sync_copy(rows_v, acc_sh.at[didx_v], add=True)

    plsc.subcore_barrier()

    @pl.loop(0, RCH)
    def _writeback(k):
        r0 = (s * RCH + k) * CH
        pltpu.sync_copy(acc_sh.at[pl.ds(r0, CH)], rows_v)
        pltpu.sync_copy(rows_v, out_hbm.at[c, pl.ds(r0, CH)])


def _fin_body(p0_ref, p1_ref, g_ref, s_ref, m_ref, b_ref, o_ref):
    acc = p0_ref[...] + p1_ref[...] + g_ref[...]
    o_ref[...] = s_ref[...] * acc + m_ref[...] * b_ref[...]


_final = pl.pallas_call(
    _fin_body,
    grid=(N // _RB,),
    in_specs=[
        pl.BlockSpec((_RB, D), lambda i: (i, 0)),
        pl.BlockSpec((_RB, D), lambda i: (i, 0)),
        pl.BlockSpec((_RB, D), lambda i: (i, 0)),
        pl.BlockSpec((_RB, 1), lambda i: (i, 0)),
        pl.BlockSpec((_RB, 1), lambda i: (i, 0)),
        pl.BlockSpec((1, D), lambda i: (0, 0)),
    ],
    out_specs=pl.BlockSpec((_RB, D), lambda i: (i, 0)),
    out_shape=jax.ShapeDtypeStruct((N, D), jnp.float32),
)


def kernel(x, edge_index, mask, W, b):
    src = edge_index[0]
    dst = edge_index[1]
    mask_f = mask.astype(jnp.float32).reshape(N, 1)
    hist = _deg_kernel(dst)
    h0 = hist[0, :N, 0:1]
    h1 = hist[1, :N, 0:1]
    g, s = _linear(x, mask_f, h0, h1, W)
    p = _edge_kernel(src, dst, g)
    return _final(p[0, :N], p[1, :N], g, s, mask_f, b.reshape(1, D))
